# baseline (speedup 1.0000x reference)
import jax
import jax.numpy as jnp
from jax import lax
from jax.experimental import pallas as pl

_NUM_CLASSES = 1000
_ROWS_PER_BLOCK = 512


def _one_hot_block(idx_ref, out_ref):
    idx = idx_ref[...]
    iota = lax.broadcasted_iota(jnp.int32, (_ROWS_PER_BLOCK, _NUM_CLASSES), 1)
    out_ref[...] = (idx == iota).astype(jnp.int32)


def kernel(x1):
    n_rows = x1.shape[0] * x1.shape[1]
    flat = x1.reshape(n_rows, 1).astype(jnp.int32)
    grid = n_rows // _ROWS_PER_BLOCK
    out = pl.pallas_call(
        _one_hot_block,
        grid=(grid,),
        in_specs=[pl.BlockSpec((_ROWS_PER_BLOCK, 1), lambda i: (i, 0))],
        out_specs=pl.BlockSpec((_ROWS_PER_BLOCK, _NUM_CLASSES), lambda i: (i, 0)),
        out_shape=jax.ShapeDtypeStruct((n_rows, _NUM_CLASSES), jnp.int32),
    )(flat)
    return out.reshape(x1.shape[0], x1.shape[1], _NUM_CLASSES)


# speedup vs baseline: 1.5546x; 1.5546x over previous
import jax
import jax.numpy as jnp
from jax import lax
from jax.experimental import pallas as pl

_NUM_CLASSES = 1000
_B = 128


def _one_hot_block(idx_ref, out_ref):
    idx = idx_ref[...]
    iota = lax.broadcasted_iota(
        jnp.int32, (_B, idx.shape[1], _NUM_CLASSES), 2)
    out_ref[...] = (idx[:, :, None] == iota).astype(jnp.int32)


def kernel(x1):
    n0, n1 = x1.shape
    x1 = x1.astype(jnp.int32)
    out = pl.pallas_call(
        _one_hot_block,
        grid=(n0 // _B,),
        in_specs=[pl.BlockSpec((_B, n1), lambda i: (i, 0))],
        out_specs=pl.BlockSpec((_B, n1, _NUM_CLASSES), lambda i: (i, 0, 0)),
        out_shape=jax.ShapeDtypeStruct((n0, n1, _NUM_CLASSES), jnp.int32),
    )(x1)
    return out
